# baseline (speedup 1.0000x reference)
import functools

import jax
import jax.numpy as jnp
import numpy as np
from jax import lax
from jax.experimental import pallas as pl
from jax.experimental.pallas import tpu as pltpu
from jax.experimental.pallas import tpu_sc as plsc

_DEPTH = 64
_SEQ = 200
_BATCH = 1024
_NC, _NS, _L = 2, 16, 16
_NW = _NC * _NS
_BBLK = _BATCH // 128
_SGRP = _NW // _BBLK
_SPW = _SEQ // _SGRP


def _pos_encoding(length, depth, n=10000):
    positions = np.arange(length)[:, np.newaxis]
    depths = np.arange(depth)[np.newaxis, :] / depth
    angle_rates = 1 / n**depths
    angle_rads = positions * angle_rates
    angle_rads[:, 0::2] = np.sin(angle_rads[:, 0::2])
    angle_rads[:, 1::2] = np.cos(angle_rads[:, 1::2])
    return angle_rads.astype(np.float32)


_POS16 = np.repeat(
    _pos_encoding(_SEQ, _DEPTH), 16, axis=1).reshape(_SEQ, 8, 128)

_IOTA = np.arange(16, dtype=np.int32)
_SKEW = [(_IOTA + k) % 16 for k in range(16)]
_UNSKEW = [(m - _IOTA) % 16 for m in range(16)]


def _make_sc_kernel():
    mesh = plsc.VectorSubcoreMesh(
        core_axis_name="c", subcore_axis_name="s", num_cores=_NC,
        num_subcores=_NS)

    @functools.partial(
        pl.kernel,
        out_type=jax.ShapeDtypeStruct((_SEQ, _DEPTH, _BATCH), jnp.float32),
        mesh=mesh,
        scratch_types=[
            pltpu.VMEM((2, 128), jnp.int32),
            pltpu.VMEM((2, 128), jnp.int32),
            pltpu.VMEM((2, 8, 128), jnp.float32),
            pltpu.VMEM((2, 128, 128), jnp.float32),
            pltpu.VMEM((2, _DEPTH, 128), jnp.float32),
            pltpu.VMEM((16, 18), jnp.float32),
        ] + [pltpu.SemaphoreType.DMA] * 8,
        compiler_params=pltpu.CompilerParams(needs_layout_passes=False),
    )
    def k(xt_hbm, table_hbm, pos_hbm, out_hbm, xrow, x2row, prow, rows,
          obuf, skew, *sems):
        xsems, psems, gsems, osems = (
            sems[0:2], sems[2:4], sems[4:6], sems[6:8])
        wid = lax.axis_index("s") * _NC + lax.axis_index("c")
        jb = (wid % _BBLK) * 128
        s0 = (wid // _BBLK) * _SPW

        def stage_start(c, ph):
            pltpu.async_copy(
                xt_hbm.at[s0 + c, pl.ds(jb, 128)], xrow.at[ph], xsems[ph])
            pltpu.async_copy(pos_hbm.at[s0 + c], prow.at[ph], psems[ph])

        def gather_start(ph):
            pltpu.async_copy(
                table_hbm.at[x2row.at[ph]], rows.at[ph], gsems[ph])

        def x2_compute(ph):
            pltpu.make_async_copy(
                xt_hbm.at[0, pl.ds(0, 128)], xrow.at[ph], xsems[ph]).wait()
            for g in range(8):
                sl = pl.ds(g * 16, 16)
                x2row[ph, sl] = jax.lax.shift_right_logical(xrow[ph, sl], 1)

        stage_start(0, 0)
        x2_compute(0)
        pltpu.make_async_copy(
            pos_hbm.at[s0], prow.at[0], psems[0]).wait()
        gather_start(0)

        iota16 = lax.iota(jnp.int32, 16)
        skews = [(iota16 + kk) & 15 for kk in range(16)]
        unskews = [(16 + m - iota16) & 15 for m in range(16)]

        @pl.loop(0, _SPW, step=2)
        def _positions(c0):
            for ph in range(2):
                c = c0 + ph
                oph = 1 - ph
                @pl.when(c + 1 < _SPW)
                def _():
                    stage_start(c + 1, oph)
                    x2_compute(oph)
                    @pl.when(c >= 1)
                    def _():
                        pltpu.make_async_copy(
                            obuf.at[oph], out_hbm.at[0, :, pl.ds(0, 128)],
                            osems[oph]).wait()
                    gather_start(oph)

                pltpu.make_async_copy(
                    table_hbm.at[x2row.at[ph]], rows.at[ph], gsems[ph]
                ).wait()
                @pl.when(c >= 1)
                def _():
                    pltpu.make_async_copy(
                        pos_hbm.at[s0], prow.at[ph], psems[ph]).wait()

                for g in range(8):
                    sl = pl.ds(g * 16, 16)
                    xv = xrow[ph, sl]
                    sel6 = (xv & 1) * 64
                    brow = iota16 + (g * 16)

                    @pl.loop(0, _DEPTH // 16)
                    def _dgroup(dg):
                        d0 = dg * 16
                        for kk in range(16):
                            col = sel6 + (skews[kk] + d0)
                            v = plsc.load_gather(rows.at[ph], [brow, col])
                            skew[kk, pl.ds(0, 16)] = v
                        for m in range(16):
                            e = plsc.load_gather(skew, [unskews[m], iota16])
                            pb = prow[ph, dg * 2 + m // 8,
                                      pl.ds((m % 8) * 16, 16)]
                            obuf[ph, d0 + m, sl] = e * 8.0 + pb

                pltpu.async_copy(
                    obuf.at[ph],
                    out_hbm.at[s0 + c, :, pl.ds(jb, 128)], osems[ph])

        for ph in range(2):
            pltpu.make_async_copy(
                obuf.at[ph], out_hbm.at[0, :, pl.ds(0, 128)],
                osems[ph]).wait()

    return k


@jax.jit
def kernel(x, table):
    table128 = table.reshape(table.shape[0] // 2, 2 * _DEPTH)
    pos16 = jnp.asarray(_POS16)
    out = _make_sc_kernel()(x.T, table128, pos16)
    return out.transpose(2, 0, 1)


# speedup vs baseline: 1.3130x; 1.3130x over previous
import functools

import jax
import jax.numpy as jnp
import numpy as np
from jax import lax
from jax.experimental import pallas as pl
from jax.experimental.pallas import tpu as pltpu
from jax.experimental.pallas import tpu_sc as plsc

_DEPTH = 64
_SEQ = 200
_NC, _NS, _L = 2, 16, 16
_NW = _NC * _NS
_CHUNK = _SEQ
_NBUF = 4


def _pos_encoding(length, depth, n=10000):
    positions = np.arange(length)[:, np.newaxis]
    depths = np.arange(depth)[np.newaxis, :] / depth
    angle_rates = 1 / n**depths
    angle_rads = positions * angle_rates
    angle_rads[:, 0::2] = np.sin(angle_rads[:, 0::2])
    angle_rads[:, 1::2] = np.cos(angle_rads[:, 1::2])
    return angle_rads.astype(np.float32)


_POS = _pos_encoding(_SEQ, _DEPTH)


def _make_sc_kernel(batch, seq):
    rows_per_w = batch * seq // _NW
    batch_per_w = batch // _NW
    n_chunks = rows_per_w // _CHUNK
    mesh = plsc.VectorSubcoreMesh(
        core_axis_name="c", subcore_axis_name="s", num_cores=_NC,
        num_subcores=_NS)

    @functools.partial(
        pl.kernel,
        out_type=jax.ShapeDtypeStruct((batch, seq, _DEPTH), jnp.float32),
        mesh=mesh,
        scratch_types=[
            pltpu.VMEM((batch_per_w, seq), jnp.int32),
            pltpu.VMEM((_SEQ, _DEPTH), jnp.float32),
            pltpu.VMEM((_NBUF, _CHUNK, _DEPTH), jnp.float32),
        ] + [pltpu.SemaphoreType.DMA] * (2 * _NBUF),
        compiler_params=pltpu.CompilerParams(use_tc_tiling_on_sc=False),
    )
    def k(x_hbm, table_hbm, pos_hbm, out_hbm, idxs, posb, rows, *sems):
        gsems, osems = sems[:_NBUF], sems[_NBUF:]
        wid = lax.axis_index("s") * _NC + lax.axis_index("c")
        b0 = wid * batch_per_w
        pltpu.sync_copy(x_hbm.at[pl.ds(b0, batch_per_w)], idxs)
        pltpu.sync_copy(pos_hbm, posb)

        def gather_start(c, b):
            pltpu.async_copy(
                table_hbm.at[idxs.at[c]], rows.at[b], gsems[b])

        def gather_wait(b):
            pltpu.make_async_copy(
                out_hbm.at[b0], rows.at[b], gsems[b]).wait()

        def scatter_wait(b):
            pltpu.make_async_copy(
                rows.at[b], out_hbm.at[b0], osems[b]).wait()

        for b in range(_NBUF - 1):
            gather_start(b, b)

        @pl.loop(0, n_chunks, step=_NBUF)
        def _chunks(c0):
            for b in range(_NBUF):
                c = c0 + b
                gather_wait(b)

                @plsc.parallel_loop(0, _CHUNK, 1, unroll=2)
                def _row(i):
                    for d in range(_DEPTH // _L):
                        sl = pl.ds(d * _L, _L)
                        rows[b, i, sl] = rows[b, i, sl] * 8.0 + posb[i, sl]

                pltpu.async_copy(
                    rows.at[b], out_hbm.at[b0 + c], osems[b])

                nc = c + _NBUF - 1
                bb = (b + _NBUF - 1) % _NBUF
                @pl.when(nc < n_chunks)
                def _():
                    @pl.when(nc >= _NBUF)
                    def _():
                        scatter_wait(bb)
                    gather_start(nc, bb)

        for b in range(_NBUF):
            scatter_wait(b)

    return k


@jax.jit
def kernel(x, table):
    b, s = x.shape
    pos = jnp.asarray(_POS)
    return _make_sc_kernel(b, s)(x.astype(jnp.int32), table, pos)
